# baseline (speedup 1.0000x reference)
import jax
import jax.numpy as jnp
from jax.experimental import pallas as pl
from jax.experimental.pallas import tpu as pltpu

_BM = 400
_HM = _BM // 2


def _gcn_kernel(adj_t_ref, adj_b_ref, x_ref, w1_ref, b1_ref, w2_ref, b2_ref,
                out_ref, h_ref):
    p = pl.program_id(0)
    i = pl.program_id(1)

    @pl.when(p == 0)
    def _layer1():
        for half, ref in ((0, adj_t_ref), (1, adj_b_ref)):
            g = jnp.dot(ref[...], x_ref[...],
                        preferred_element_type=jnp.float32)
            h = jnp.dot(g, w1_ref[...],
                        preferred_element_type=jnp.float32) + b1_ref[...]
            h_ref[pl.ds(i * _BM + half * _HM, _HM), :] = jnp.maximum(h, 0.0)

    @pl.when(p == 1)
    def _layer2():
        for half, ref in ((0, adj_t_ref), (1, adj_b_ref)):
            g = jnp.dot(ref[...], h_ref[...],
                        preferred_element_type=jnp.float32)
            out_ref[pl.ds(half * _HM, _HM), :] = jnp.dot(
                g, w2_ref[...],
                preferred_element_type=jnp.float32) + b2_ref[...]


def kernel(x, adj, W1, b1, W2, b2):
    n, d = x.shape
    nb = n // _BM
    return pl.pallas_call(
        _gcn_kernel,
        grid=(2, nb),
        in_specs=[
            pl.BlockSpec((_HM, n), lambda p, i: (2 * i, 0)),
            pl.BlockSpec((_HM, n), lambda p, i: (2 * i + 1, 0)),
            pl.BlockSpec((n, d), lambda p, i: (0, 0)),
            pl.BlockSpec((d, d), lambda p, i: (0, 0)),
            pl.BlockSpec((1, d), lambda p, i: (0, 0)),
            pl.BlockSpec((d, d), lambda p, i: (0, 0)),
            pl.BlockSpec((1, d), lambda p, i: (0, 0)),
        ],
        out_specs=pl.BlockSpec((_BM, d), lambda p, i: (i * p, 0)),
        out_shape=jax.ShapeDtypeStruct((n, d), jnp.float32),
        scratch_shapes=[pltpu.VMEM((n, d), jnp.float32)],
    )(adj, adj, x, W1, b1.reshape(1, -1), W2, b2.reshape(1, -1))


# speedup vs baseline: 1.0640x; 1.0640x over previous
import functools

import jax
import jax.numpy as jnp
from jax.experimental import pallas as pl
from jax.experimental.pallas import tpu as pltpu

_BM = 200
_NBUF = 4


def _gcn_kernel(adj_hbm, x_ref, w1_ref, b1_ref, w2_ref, b2_ref, out_ref,
                ring, h_ref, sems, *, nb):
    total = 2 * nb

    def _start(s):
        r = jax.lax.rem(s, nb)
        b = jax.lax.rem(s, _NBUF)
        pltpu.make_async_copy(
            adj_hbm.at[pl.ds(r * _BM, _BM), :], ring.at[b], sems.at[b]
        ).start()

    for s in range(_NBUF):
        _start(s)

    def _step(s, carry):
        b = jax.lax.rem(s, _NBUF)
        r = jax.lax.rem(s, nb)
        pltpu.make_async_copy(
            adj_hbm.at[pl.ds(r * _BM, _BM), :], ring.at[b], sems.at[b]
        ).wait()
        adj_blk = ring[b]

        @pl.when(s < nb)
        def _layer1():
            g = jnp.dot(adj_blk, x_ref[...],
                        preferred_element_type=jnp.float32)
            h = jnp.dot(g, w1_ref[...],
                        preferred_element_type=jnp.float32) + b1_ref[...]
            h_ref[pl.ds(r * _BM, _BM), :] = jnp.maximum(h, 0.0)

        @pl.when(s >= nb)
        def _layer2():
            g = jnp.dot(adj_blk, h_ref[...],
                        preferred_element_type=jnp.float32)
            out_ref[pl.ds(r * _BM, _BM), :] = jnp.dot(
                g, w2_ref[...],
                preferred_element_type=jnp.float32) + b2_ref[...]

        @pl.when(s + _NBUF < total)
        def _prefetch():
            _start(s + _NBUF)

        return carry

    jax.lax.fori_loop(0, total, _step, 0)


def kernel(x, adj, W1, b1, W2, b2):
    n, d = x.shape
    nb = n // _BM
    return pl.pallas_call(
        functools.partial(_gcn_kernel, nb=nb),
        in_specs=[
            pl.BlockSpec(memory_space=pl.ANY),
            pl.BlockSpec(memory_space=pltpu.MemorySpace.VMEM),
            pl.BlockSpec(memory_space=pltpu.MemorySpace.VMEM),
            pl.BlockSpec(memory_space=pltpu.MemorySpace.VMEM),
            pl.BlockSpec(memory_space=pltpu.MemorySpace.VMEM),
            pl.BlockSpec(memory_space=pltpu.MemorySpace.VMEM),
        ],
        out_specs=pl.BlockSpec(memory_space=pltpu.MemorySpace.VMEM),
        out_shape=jax.ShapeDtypeStruct((n, d), jnp.float32),
        scratch_shapes=[
            pltpu.VMEM((_NBUF, _BM, n), jnp.float32),
            pltpu.VMEM((n, d), jnp.float32),
            pltpu.SemaphoreType.DMA((_NBUF,)),
        ],
    )(adj, x, W1, b1.reshape(1, -1), W2, b2.reshape(1, -1))
